# baseline (speedup 1.0000x reference)
import functools

import jax
import jax.numpy as jnp
from jax import lax
from jax.experimental import pallas as pl
from jax.experimental.pallas import tpu as pltpu
from jax.experimental.pallas import tpu_sc as plsc

NU = 50000
NJ = 50000
EDGES = 800000
DF = 128
DE = 64
DH = 64
BSEL = 4096
HALF = 32

NSC = 2
NTILE = 16
W = 128
KSUB = 2
SUPS = 196
MEGA = 14
NMEGA = SUPS // MEGA
IDXR = MEGA * KSUB
TILE_E = SUPS * KSUB * W
EPAD = NTILE * TILE_E
EROWS = EPAD // W
NPAD = 50176
RPT = NPAD // NTILE
WB = RPT // 4

_MESH = plsc.VectorSubcoreMesh(core_axis_name="c", subcore_axis_name="s",
                               num_cores=NSC, num_subcores=NTILE)


def _embed_block(x_ref, w_ref, b_ref, h0_ref, h1_ref):
    h = jnp.dot(x_ref[...], w_ref[...], preferred_element_type=jnp.float32)
    h = h + b_ref[...]
    h0_ref[...] = h[:, :HALF]
    h1_ref[...] = h[:, HALF:]


def _embed(x, w, b, rows=1000):
    n = x.shape[0]
    return pl.pallas_call(
        _embed_block,
        grid=(n // rows,),
        in_specs=[pl.BlockSpec((rows, DF), lambda i: (i, 0)),
                  pl.BlockSpec((DF, DE), lambda i: (0, 0)),
                  pl.BlockSpec((1, DE), lambda i: (0, 0))],
        out_specs=[pl.BlockSpec((rows, HALF), lambda i: (i, 0)),
                   pl.BlockSpec((rows, HALF), lambda i: (i, 0))],
        out_shape=[jax.ShapeDtypeStruct((n, HALF), jnp.float32)] * 2,
    )(x, w, b.reshape(1, DE))


def _combine_block(a0_ref, a1_ref, d_ref, h0_ref, h1_ref,
                   wl_ref, bl_ref, wr_ref, o0_ref, o1_ref):
    r = 1.0 / jnp.maximum(d_ref[...], 1.0)
    a = jnp.concatenate([a0_ref[...] * r, a1_ref[...] * r], axis=1)
    h = jnp.concatenate([h0_ref[...], h1_ref[...]], axis=1)
    o = jnp.dot(a, wl_ref[...], preferred_element_type=jnp.float32)
    o = o + bl_ref[...]
    o = o + jnp.dot(h, wr_ref[...], preferred_element_type=jnp.float32)
    o = jnp.maximum(o, 0.0)
    o0_ref[...] = o[:, :HALF]
    o1_ref[...] = o[:, HALF:]


def _combine(a0, a1, deg, h0, h1, wl, bl, wr, rows=1000):
    n = h0.shape[0]
    return pl.pallas_call(
        _combine_block,
        grid=(n // rows,),
        in_specs=[pl.BlockSpec((rows, HALF), lambda i: (i, 0)),
                  pl.BlockSpec((rows, HALF), lambda i: (i, 0)),
                  pl.BlockSpec((rows, 1), lambda i: (i, 0)),
                  pl.BlockSpec((rows, HALF), lambda i: (i, 0)),
                  pl.BlockSpec((rows, HALF), lambda i: (i, 0)),
                  pl.BlockSpec((DE, DH), lambda i: (0, 0)),
                  pl.BlockSpec((1, DH), lambda i: (0, 0)),
                  pl.BlockSpec((DE, DH), lambda i: (0, 0))],
        out_specs=[pl.BlockSpec((rows, HALF), lambda i: (i, 0)),
                   pl.BlockSpec((rows, HALF), lambda i: (i, 0))],
        out_shape=[jax.ShapeDtypeStruct((n, HALF), jnp.float32)] * 2,
    )(a0, a1, deg.reshape(n, 1) if deg.shape[0] == n else deg[:n].reshape(n, 1),
      h0, h1, wl, bl.reshape(1, DH), wr)


def _pred_block(au0, au1, du, ue0, ue1, aj0, aj1, dj, je0, je1,
                vp_ref, c_ref, o_ref):
    vp = vp_ref[...]
    ru = 1.0 / jnp.maximum(du[...], 1.0)
    rj = 1.0 / jnp.maximum(dj[...], 1.0)
    s = jnp.sum(au0[...] * vp[0:1] + au1[...] * vp[1:2], axis=1, keepdims=True) * ru
    s = s + jnp.sum(ue0[...] * vp[2:3] + ue1[...] * vp[3:4], axis=1, keepdims=True)
    s = s + jnp.sum(aj0[...] * vp[4:5] + aj1[...] * vp[5:6], axis=1, keepdims=True) * rj
    s = s + jnp.sum(je0[...] * vp[6:7] + je1[...] * vp[7:8], axis=1, keepdims=True)
    o_ref[...] = s + c_ref[...]


def _pred(au0, au1, du, ue0, ue1, aj0, aj1, dj, je0, je1, vpack, c):
    n = au0.shape[0]
    half_spec = pl.BlockSpec((n, HALF), lambda: (0, 0))
    one_spec = pl.BlockSpec((n, 1), lambda: (0, 0))
    return pl.pallas_call(
        _pred_block,
        in_specs=[half_spec, half_spec, one_spec, half_spec, half_spec,
                  half_spec, half_spec, one_spec, half_spec, half_spec,
                  pl.BlockSpec((8, HALF), lambda: (0, 0)),
                  pl.BlockSpec((1, 1), lambda: (0, 0))],
        out_specs=pl.BlockSpec((n, 1), lambda: (0, 0)),
        out_shape=jax.ShapeDtypeStruct((n, 1), jnp.float32),
    )(au0, au1, du.reshape(n, 1), ue0, ue1, aj0, aj1, dj.reshape(n, 1),
      je0, je1, vpack, c.reshape(1, 1))


def _segsum_body(with_deg, *refs):
    if with_deg:
        (t0h, t1h, idxg_h, idxs_h, idxd_h, zrow_h, zdeg_h,
         o0, o1, dj_o, du_o,
         ig_v, is_v, rows_v, acc_sh, sem, id_v, ones_v, dv_v, deg_sh) = refs
    else:
        (t0h, t1h, idxg_h, idxs_h, zrow_h,
         o0, o1,
         ig_v, is_v, rows_v, acc_sh, sem) = refs
    c = lax.axis_index("c")
    t = lax.axis_index("s")

    for i in range(4):
        pltpu.sync_copy(zrow_h, acc_sh.at[pl.ds(t * RPT + i * WB, WB)])
    if with_deg:
        pltpu.sync_copy(zdeg_h, dv_v)
        pltpu.sync_copy(dv_v, deg_sh.at[pl.ds(t * RPT, RPT)])
        for i in range(W // 16):
            ones_v[pl.ds(i * 16, 16)] = jnp.full((16,), 1.0, jnp.float32)
    plsc.subcore_barrier()

    tbase = t * (SUPS * KSUB)

    def super_body(s, carry):
        rowbase = tbase + s * KSUB
        pltpu.sync_copy(idxg_h.at[pl.ds(rowbase, KSUB)], ig_v)
        pltpu.sync_copy(idxs_h.at[pl.ds(rowbase, KSUB)], is_v)
        if with_deg:
            pltpu.sync_copy(idxd_h.at[pl.ds(c * EROWS + rowbase, KSUB)], id_v)

        def gather_from(tab):
            def go():
                descs = [pltpu.async_copy(tab.at[ig_v.at[j]],
                                          rows_v.at[pl.ds(j * W, W)], sem)
                         for j in range(KSUB)]
                for d in descs:
                    d.wait()
            return go
        pl.when(c == 0)(gather_from(t0h))
        pl.when(c == 1)(gather_from(t1h))

        for j in range(KSUB):
            pltpu.sync_copy(rows_v.at[pl.ds(j * W, W)],
                            acc_sh.at[is_v.at[j]], add=True)
        if with_deg:
            for j in range(KSUB):
                pltpu.sync_copy(ones_v, deg_sh.at[id_v.at[j]], add=True)
        return carry

    lax.fori_loop(0, SUPS, super_body, 0)
    plsc.subcore_barrier()

    def wb(dst):
        def go():
            for i in range(4):
                r0 = t * RPT + i * WB
                pltpu.sync_copy(acc_sh.at[pl.ds(r0, WB)], dst.at[pl.ds(r0, WB)])
        return go
    pl.when(c == 0)(wb(o0))
    pl.when(c == 1)(wb(o1))
    if with_deg:
        def wbd(dst):
            def go():
                pltpu.sync_copy(deg_sh.at[pl.ds(t * RPT, RPT)], dv_v)
                pltpu.sync_copy(dv_v, dst.at[pl.ds(t * RPT, RPT)])
            return go
        pl.when(c == 0)(wbd(dj_o))
        pl.when(c == 1)(wbd(du_o))


def _make_segsum(with_deg):
    out_type = [jax.ShapeDtypeStruct((NPAD, HALF), jnp.float32)] * 2
    scratch = [pltpu.VMEM((KSUB, W), jnp.int32),
               pltpu.VMEM((KSUB, W), jnp.int32),
               pltpu.VMEM((KSUB * W, HALF), jnp.float32),
               pltpu.VMEM_SHARED((NPAD, HALF), jnp.float32),
               pltpu.SemaphoreType.DMA]
    if with_deg:
        out_type = out_type + [jax.ShapeDtypeStruct((NPAD,), jnp.float32)] * 2
        scratch = scratch + [pltpu.VMEM((KSUB, W), jnp.int32),
                             pltpu.VMEM((W,), jnp.float32),
                             pltpu.VMEM((RPT,), jnp.float32),
                             pltpu.VMEM_SHARED((NPAD,), jnp.float32)]
    return pl.kernel(functools.partial(_segsum_body, with_deg),
                     out_type=out_type, mesh=_MESH, scratch_types=scratch,
                     compiler_params=pltpu.CompilerParams(
                         use_tc_tiling_on_sc=False))


_segsum = _make_segsum(False)
_segsum_deg = _make_segsum(True)


def _gathersel_body(*refs):
    (au0h, au1h, nu0h, nu1h, aj0h, aj1h, nj0h, nj1h, du_h, dj_h, ui_h, ji_h,
     oau0, oau1, onu0, onu1, oaj0, oaj1, onj0, onj1, odu, odj,
     iu_v, ij_v, b0, b1, b2, b3, b4, b5, b6, b7, du_v, dj_v, sem) = refs
    c = lax.axis_index("c")
    t = lax.axis_index("s")
    base = (t * NSC + c) * W
    pltpu.sync_copy(ui_h.at[pl.ds(base, W)], iu_v)
    pltpu.sync_copy(ji_h.at[pl.ds(base, W)], ij_v)
    rows = [(au0h, b0, oau0, iu_v), (au1h, b1, oau1, iu_v),
            (nu0h, b2, onu0, iu_v), (nu1h, b3, onu1, iu_v),
            (aj0h, b4, oaj0, ij_v), (aj1h, b5, oaj1, ij_v),
            (nj0h, b6, onj0, ij_v), (nj1h, b7, onj1, ij_v)]
    descs = [pltpu.async_copy(tab.at[iv], buf, sem)
             for tab, buf, _, iv in rows]
    descs.append(pltpu.async_copy(du_h.at[iu_v], du_v, sem))
    descs.append(pltpu.async_copy(dj_h.at[ij_v], dj_v, sem))
    for d in descs:
        d.wait()
    for _, buf, out, _ in rows:
        pltpu.sync_copy(buf, out.at[pl.ds(base, W)])
    pltpu.sync_copy(du_v, odu.at[pl.ds(base, W)])
    pltpu.sync_copy(dj_v, odj.at[pl.ds(base, W)])


_gathersel = pl.kernel(
    _gathersel_body,
    out_type=[jax.ShapeDtypeStruct((BSEL, HALF), jnp.float32)] * 8
             + [jax.ShapeDtypeStruct((BSEL,), jnp.float32)] * 2,
    mesh=_MESH,
    scratch_types=[pltpu.VMEM((W,), jnp.int32)] * 2
                  + [pltpu.VMEM((W, HALF), jnp.float32)] * 8
                  + [pltpu.VMEM((W,), jnp.float32)] * 2
                  + [pltpu.SemaphoreType.DMA],
    compiler_params=pltpu.CompilerParams(use_tc_tiling_on_sc=False))


def kernel(x_user, x_job, edge_index, user_indices, job_indices,
           W_emb_user, b_emb_user, W_emb_job, b_emb_job,
           Wl0_uj, bl0_uj, Wr0_uj, Wl0_ju, bl0_ju, Wr0_ju,
           Wl1_uj, bl1_uj, Wr1_uj, Wl1_ju, bl1_ju, Wr1_ju,
           W_pred, b_pred):
    src = edge_index[0]
    dst = edge_index[1]

    hu0, hu1 = _embed(x_user, W_emb_user, b_emb_user)
    hj0, hj1 = _embed(x_job, W_emb_job, b_emb_job)

    pad_n = EPAD - EDGES
    ar = jnp.arange(pad_n, dtype=jnp.int32)
    pad_g = ar % 128
    pad_s = NJ + ar % (NPAD - NJ)
    src_g = jnp.concatenate([src, pad_g]).reshape(EROWS, W)
    dst_s = jnp.concatenate([dst, pad_s]).reshape(EROWS, W)
    dst_g = jnp.concatenate([dst, pad_g]).reshape(EROWS, W)
    src_s = jnp.concatenate([src, pad_s]).reshape(EROWS, W)
    idxd = jnp.concatenate([dst_s, src_s], axis=0)
    zrow = jnp.zeros((WB, HALF), jnp.float32)
    zdeg = jnp.zeros((RPT,), jnp.float32)

    aj0, aj1, deg_j, deg_u = _segsum_deg(hu0, hu1, src_g, dst_s, idxd,
                                         zrow, zdeg)
    au0, au1 = _segsum(hj0, hj1, dst_g, src_s, zrow)
    nj0, nj1 = _combine(aj0, aj1, deg_j, hj0, hj1, Wl0_uj, bl0_uj, Wr0_uj)
    nu0, nu1 = _combine(au0, au1, deg_u, hu0, hu1, Wl0_ju, bl0_ju, Wr0_ju)

    aj0, aj1 = _segsum(nu0, nu1, src_g, dst_s, zrow)
    au0, au1 = _segsum(nj0, nj1, dst_g, src_s, zrow)

    wp_u = W_pred[:DH, 0]
    wp_j = W_pred[DH:, 0]
    v1 = Wl1_ju @ wp_u
    v2 = Wr1_ju @ wp_u
    v3 = Wl1_uj @ wp_j
    v4 = Wr1_uj @ wp_j
    c = bl1_ju @ wp_u + bl1_uj @ wp_j + b_pred[0]
    vpack = jnp.stack([v1[:HALF], v1[HALF:], v2[:HALF], v2[HALF:],
                       v3[:HALF], v3[HALF:], v4[:HALF], v4[HALF:]], axis=0)

    (au0s, au1s, ue0, ue1, aj0s, aj1s, je0, je1, dus, djs) = _gathersel(
        au0, au1, nu0, nu1, aj0, aj1, nj0, nj1, deg_u, deg_j,
        user_indices, job_indices)

    preds = _pred(au0s, au1s, dus, ue0, ue1, aj0s, aj1s, djs, je0, je1,
                  vpack, c)
    return preds[:, 0]


# speedup vs baseline: 1.3748x; 1.3748x over previous
import functools

import jax
import jax.numpy as jnp
from jax import lax
from jax.experimental import pallas as pl
from jax.experimental.pallas import tpu as pltpu
from jax.experimental.pallas import tpu_sc as plsc

NU = 50000
NJ = 50000
EDGES = 800000
DF = 128
DE = 64
DH = 64
BSEL = 4096
HALF = 32

NSC = 2
NTILE = 16
W = 128
KSUB = 4
SUPS = 98
TILE_E = SUPS * KSUB * W
EPAD = NTILE * TILE_E
EROWS = EPAD // W
NPAD = 50176
RPT = NPAD // NTILE
WB = RPT // 4

_MESH = plsc.VectorSubcoreMesh(core_axis_name="c", subcore_axis_name="s",
                               num_cores=NSC, num_subcores=NTILE)


def _embed_block(x_ref, w_ref, b_ref, h0_ref, h1_ref):
    h = jnp.dot(x_ref[...], w_ref[...], preferred_element_type=jnp.float32)
    h = h + b_ref[...]
    h0_ref[...] = h[:, :HALF]
    h1_ref[...] = h[:, HALF:]


def _embed(x, w, b, rows=1000):
    n = x.shape[0]
    return pl.pallas_call(
        _embed_block,
        grid=(n // rows,),
        in_specs=[pl.BlockSpec((rows, DF), lambda i: (i, 0)),
                  pl.BlockSpec((DF, DE), lambda i: (0, 0)),
                  pl.BlockSpec((1, DE), lambda i: (0, 0))],
        out_specs=[pl.BlockSpec((rows, HALF), lambda i: (i, 0)),
                   pl.BlockSpec((rows, HALF), lambda i: (i, 0))],
        out_shape=[jax.ShapeDtypeStruct((n, HALF), jnp.float32)] * 2,
    )(x, w, b.reshape(1, DE))


def _combine_block(a0_ref, a1_ref, d_ref, h0_ref, h1_ref,
                   wl_ref, bl_ref, wr_ref, o0_ref, o1_ref):
    r = 1.0 / jnp.maximum(d_ref[...], 1.0)
    a = jnp.concatenate([a0_ref[...] * r, a1_ref[...] * r], axis=1)
    h = jnp.concatenate([h0_ref[...], h1_ref[...]], axis=1)
    o = jnp.dot(a, wl_ref[...], preferred_element_type=jnp.float32)
    o = o + bl_ref[...]
    o = o + jnp.dot(h, wr_ref[...], preferred_element_type=jnp.float32)
    o = jnp.maximum(o, 0.0)
    o0_ref[...] = o[:, :HALF]
    o1_ref[...] = o[:, HALF:]


def _combine(a0, a1, deg, h0, h1, wl, bl, wr, rows=1000):
    n = h0.shape[0]
    return pl.pallas_call(
        _combine_block,
        grid=(n // rows,),
        in_specs=[pl.BlockSpec((rows, HALF), lambda i: (i, 0)),
                  pl.BlockSpec((rows, HALF), lambda i: (i, 0)),
                  pl.BlockSpec((rows, 1), lambda i: (i, 0)),
                  pl.BlockSpec((rows, HALF), lambda i: (i, 0)),
                  pl.BlockSpec((rows, HALF), lambda i: (i, 0)),
                  pl.BlockSpec((DE, DH), lambda i: (0, 0)),
                  pl.BlockSpec((1, DH), lambda i: (0, 0)),
                  pl.BlockSpec((DE, DH), lambda i: (0, 0))],
        out_specs=[pl.BlockSpec((rows, HALF), lambda i: (i, 0)),
                   pl.BlockSpec((rows, HALF), lambda i: (i, 0))],
        out_shape=[jax.ShapeDtypeStruct((n, HALF), jnp.float32)] * 2,
    )(a0, a1, deg.reshape(n, 1) if deg.shape[0] == n else deg[:n].reshape(n, 1),
      h0, h1, wl, bl.reshape(1, DH), wr)


def _pred_block(au0, au1, du, ue0, ue1, aj0, aj1, dj, je0, je1,
                vp_ref, c_ref, o_ref):
    vp = vp_ref[...]
    ru = 1.0 / jnp.maximum(du[...], 1.0)
    rj = 1.0 / jnp.maximum(dj[...], 1.0)
    s = jnp.sum(au0[...] * vp[0:1] + au1[...] * vp[1:2], axis=1, keepdims=True) * ru
    s = s + jnp.sum(ue0[...] * vp[2:3] + ue1[...] * vp[3:4], axis=1, keepdims=True)
    s = s + jnp.sum(aj0[...] * vp[4:5] + aj1[...] * vp[5:6], axis=1, keepdims=True) * rj
    s = s + jnp.sum(je0[...] * vp[6:7] + je1[...] * vp[7:8], axis=1, keepdims=True)
    o_ref[...] = s + c_ref[...]


def _pred(au0, au1, du, ue0, ue1, aj0, aj1, dj, je0, je1, vpack, c):
    n = au0.shape[0]
    half_spec = pl.BlockSpec((n, HALF), lambda: (0, 0))
    one_spec = pl.BlockSpec((n, 1), lambda: (0, 0))
    return pl.pallas_call(
        _pred_block,
        in_specs=[half_spec, half_spec, one_spec, half_spec, half_spec,
                  half_spec, half_spec, one_spec, half_spec, half_spec,
                  pl.BlockSpec((8, HALF), lambda: (0, 0)),
                  pl.BlockSpec((1, 1), lambda: (0, 0))],
        out_specs=pl.BlockSpec((n, 1), lambda: (0, 0)),
        out_shape=jax.ShapeDtypeStruct((n, 1), jnp.float32),
    )(au0, au1, du.reshape(n, 1), ue0, ue1, aj0, aj1, dj.reshape(n, 1),
      je0, je1, vpack, c.reshape(1, 1))


def _segsum_body(with_deg, *refs):
    if with_deg:
        (t0h, t1h, idxg_h, idxs_h, idxd_h, zrow_h, zdeg_h,
         o0, o1, dj_o, du_o,
         ig_v, is_v, rows_v, acc_sh, sem, id_v, ones_v, dv_v, deg_sh) = refs
    else:
        (t0h, t1h, idxg_h, idxs_h, zrow_h,
         o0, o1,
         ig_v, is_v, rows_v, acc_sh, sem) = refs
    c = lax.axis_index("c")
    t = lax.axis_index("s")

    for i in range(4):
        pltpu.sync_copy(zrow_h, acc_sh.at[pl.ds(t * RPT + i * WB, WB)])
    if with_deg:
        pltpu.sync_copy(zdeg_h, dv_v)
        pltpu.sync_copy(dv_v, deg_sh.at[pl.ds(t * RPT, RPT)])
        for i in range(W // 16):
            ones_v[pl.ds(i * 16, 16)] = jnp.full((16,), 1.0, jnp.float32)
    plsc.subcore_barrier()

    tbase = t * (SUPS * KSUB)

    def super_body(s, carry):
        rowbase = tbase + s * KSUB
        pltpu.sync_copy(idxg_h.at[pl.ds(rowbase, KSUB)], ig_v)
        pltpu.sync_copy(idxs_h.at[pl.ds(rowbase, KSUB)], is_v)
        if with_deg:
            pltpu.sync_copy(idxd_h.at[pl.ds(c * EROWS + rowbase, KSUB)], id_v)

        def gather_from(tab):
            def go():
                descs = [pltpu.async_copy(tab.at[ig_v.at[j]],
                                          rows_v.at[pl.ds(j * W, W)], sem)
                         for j in range(KSUB)]
                for d in descs:
                    d.wait()
            return go
        pl.when(c == 0)(gather_from(t0h))
        pl.when(c == 1)(gather_from(t1h))

        for j in range(KSUB):
            pltpu.sync_copy(rows_v.at[pl.ds(j * W, W)],
                            acc_sh.at[is_v.at[j]], add=True)
        if with_deg:
            for j in range(KSUB):
                pltpu.sync_copy(ones_v, deg_sh.at[id_v.at[j]], add=True)
        return carry

    lax.fori_loop(0, SUPS, super_body, 0)
    plsc.subcore_barrier()

    def wb(dst):
        def go():
            for i in range(4):
                r0 = t * RPT + i * WB
                pltpu.sync_copy(acc_sh.at[pl.ds(r0, WB)], dst.at[pl.ds(r0, WB)])
        return go
    pl.when(c == 0)(wb(o0))
    pl.when(c == 1)(wb(o1))
    if with_deg:
        def wbd(dst):
            def go():
                pltpu.sync_copy(deg_sh.at[pl.ds(t * RPT, RPT)], dv_v)
                pltpu.sync_copy(dv_v, dst.at[pl.ds(t * RPT, RPT)])
            return go
        pl.when(c == 0)(wbd(dj_o))
        pl.when(c == 1)(wbd(du_o))


def _make_segsum(with_deg):
    out_type = [jax.ShapeDtypeStruct((NPAD, HALF), jnp.float32)] * 2
    scratch = [pltpu.VMEM((KSUB, W), jnp.int32),
               pltpu.VMEM((KSUB, W), jnp.int32),
               pltpu.VMEM((KSUB * W, HALF), jnp.float32),
               pltpu.VMEM_SHARED((NPAD, HALF), jnp.float32),
               pltpu.SemaphoreType.DMA]
    if with_deg:
        out_type = out_type + [jax.ShapeDtypeStruct((NPAD,), jnp.float32)] * 2
        scratch = scratch + [pltpu.VMEM((KSUB, W), jnp.int32),
                             pltpu.VMEM((W,), jnp.float32),
                             pltpu.VMEM((RPT,), jnp.float32),
                             pltpu.VMEM_SHARED((NPAD,), jnp.float32)]
    return pl.kernel(functools.partial(_segsum_body, with_deg),
                     out_type=out_type, mesh=_MESH, scratch_types=scratch,
                     compiler_params=pltpu.CompilerParams(
                         use_tc_tiling_on_sc=False))


_segsum = _make_segsum(False)
_segsum_deg = _make_segsum(True)


def _gathersel_body(*refs):
    (au0h, au1h, nu0h, nu1h, aj0h, aj1h, nj0h, nj1h, du_h, dj_h, ui_h, ji_h,
     oau0, oau1, onu0, onu1, oaj0, oaj1, onj0, onj1, odu, odj,
     iu_v, ij_v, b0, b1, b2, b3, b4, b5, b6, b7, du_v, dj_v, sem) = refs
    c = lax.axis_index("c")
    t = lax.axis_index("s")
    base = (t * NSC + c) * W
    pltpu.sync_copy(ui_h.at[pl.ds(base, W)], iu_v)
    pltpu.sync_copy(ji_h.at[pl.ds(base, W)], ij_v)
    rows = [(au0h, b0, oau0, iu_v), (au1h, b1, oau1, iu_v),
            (nu0h, b2, onu0, iu_v), (nu1h, b3, onu1, iu_v),
            (aj0h, b4, oaj0, ij_v), (aj1h, b5, oaj1, ij_v),
            (nj0h, b6, onj0, ij_v), (nj1h, b7, onj1, ij_v)]
    descs = [pltpu.async_copy(tab.at[iv], buf, sem)
             for tab, buf, _, iv in rows]
    descs.append(pltpu.async_copy(du_h.at[iu_v], du_v, sem))
    descs.append(pltpu.async_copy(dj_h.at[ij_v], dj_v, sem))
    for d in descs:
        d.wait()
    for _, buf, out, _ in rows:
        pltpu.sync_copy(buf, out.at[pl.ds(base, W)])
    pltpu.sync_copy(du_v, odu.at[pl.ds(base, W)])
    pltpu.sync_copy(dj_v, odj.at[pl.ds(base, W)])


_gathersel = pl.kernel(
    _gathersel_body,
    out_type=[jax.ShapeDtypeStruct((BSEL, HALF), jnp.float32)] * 8
             + [jax.ShapeDtypeStruct((BSEL,), jnp.float32)] * 2,
    mesh=_MESH,
    scratch_types=[pltpu.VMEM((W,), jnp.int32)] * 2
                  + [pltpu.VMEM((W, HALF), jnp.float32)] * 8
                  + [pltpu.VMEM((W,), jnp.float32)] * 2
                  + [pltpu.SemaphoreType.DMA],
    compiler_params=pltpu.CompilerParams(use_tc_tiling_on_sc=False))


def kernel(x_user, x_job, edge_index, user_indices, job_indices,
           W_emb_user, b_emb_user, W_emb_job, b_emb_job,
           Wl0_uj, bl0_uj, Wr0_uj, Wl0_ju, bl0_ju, Wr0_ju,
           Wl1_uj, bl1_uj, Wr1_uj, Wl1_ju, bl1_ju, Wr1_ju,
           W_pred, b_pred):
    src = edge_index[0]
    dst = edge_index[1]

    hu0, hu1 = _embed(x_user, W_emb_user, b_emb_user)
    hj0, hj1 = _embed(x_job, W_emb_job, b_emb_job)

    pad_n = EPAD - EDGES
    ar = jnp.arange(pad_n, dtype=jnp.int32)
    pad_g = ar % 128
    pad_s = NJ + ar % (NPAD - NJ)
    src_g = jnp.concatenate([src, pad_g]).reshape(EROWS, W)
    dst_s = jnp.concatenate([dst, pad_s]).reshape(EROWS, W)
    dst_g = jnp.concatenate([dst, pad_g]).reshape(EROWS, W)
    src_s = jnp.concatenate([src, pad_s]).reshape(EROWS, W)
    idxd = jnp.concatenate([dst_s, src_s], axis=0)
    zrow = jnp.zeros((WB, HALF), jnp.float32)
    zdeg = jnp.zeros((RPT,), jnp.float32)

    aj0, aj1, deg_j, deg_u = _segsum_deg(hu0, hu1, src_g, dst_s, idxd,
                                         zrow, zdeg)
    au0, au1 = _segsum(hj0, hj1, dst_g, src_s, zrow)
    nj0, nj1 = _combine(aj0, aj1, deg_j, hj0, hj1, Wl0_uj, bl0_uj, Wr0_uj)
    nu0, nu1 = _combine(au0, au1, deg_u, hu0, hu1, Wl0_ju, bl0_ju, Wr0_ju)

    aj0, aj1 = _segsum(nu0, nu1, src_g, dst_s, zrow)
    au0, au1 = _segsum(nj0, nj1, dst_g, src_s, zrow)

    wp_u = W_pred[:DH, 0]
    wp_j = W_pred[DH:, 0]
    v1 = Wl1_ju @ wp_u
    v2 = Wr1_ju @ wp_u
    v3 = Wl1_uj @ wp_j
    v4 = Wr1_uj @ wp_j
    c = bl1_ju @ wp_u + bl1_uj @ wp_j + b_pred[0]
    vpack = jnp.stack([v1[:HALF], v1[HALF:], v2[:HALF], v2[HALF:],
                       v3[:HALF], v3[HALF:], v4[:HALF], v4[HALF:]], axis=0)

    (au0s, au1s, ue0, ue1, aj0s, aj1s, je0, je1, dus, djs) = _gathersel(
        au0, au1, nu0, nu1, aj0, aj1, nj0, nj1, deg_u, deg_j,
        user_indices, job_indices)

    preds = _pred(au0s, au1s, dus, ue0, ue1, aj0s, aj1s, djs, je0, je1,
                  vpack, c)
    return preds[:, 0]


# speedup vs baseline: 1.4822x; 1.0781x over previous
import functools

import jax
import jax.numpy as jnp
from jax import lax
from jax.experimental import pallas as pl
from jax.experimental.pallas import tpu as pltpu
from jax.experimental.pallas import tpu_sc as plsc

NU = 50000
NJ = 50000
EDGES = 800000
DF = 128
DE = 64
DH = 64
BSEL = 4096
HALF = 32

NSC = 2
NTILE = 16
W = 128
KSUB = 4
SUPS = 98
KS2 = 2
NWIN = SUPS * KSUB // KS2
NGRP = NWIN // 2
TILE_E = SUPS * KSUB * W
EPAD = NTILE * TILE_E
EROWS = EPAD // W
NPAD = 50176
RPT = NPAD // NTILE
WB = RPT // 4

_MESH = plsc.VectorSubcoreMesh(core_axis_name="c", subcore_axis_name="s",
                               num_cores=NSC, num_subcores=NTILE)


def _embed_block(x_ref, w_ref, b_ref, h0_ref, h1_ref):
    h = jnp.dot(x_ref[...], w_ref[...], preferred_element_type=jnp.float32)
    h = h + b_ref[...]
    h0_ref[...] = h[:, :HALF]
    h1_ref[...] = h[:, HALF:]


def _embed(x, w, b, rows=1000):
    n = x.shape[0]
    return pl.pallas_call(
        _embed_block,
        grid=(n // rows,),
        in_specs=[pl.BlockSpec((rows, DF), lambda i: (i, 0)),
                  pl.BlockSpec((DF, DE), lambda i: (0, 0)),
                  pl.BlockSpec((1, DE), lambda i: (0, 0))],
        out_specs=[pl.BlockSpec((rows, HALF), lambda i: (i, 0)),
                   pl.BlockSpec((rows, HALF), lambda i: (i, 0))],
        out_shape=[jax.ShapeDtypeStruct((n, HALF), jnp.float32)] * 2,
    )(x, w, b.reshape(1, DE))


def _combine_block(a0_ref, a1_ref, d_ref, h0_ref, h1_ref,
                   wl_ref, bl_ref, wr_ref, o0_ref, o1_ref):
    r = 1.0 / jnp.maximum(d_ref[...], 1.0)
    a = jnp.concatenate([a0_ref[...] * r, a1_ref[...] * r], axis=1)
    h = jnp.concatenate([h0_ref[...], h1_ref[...]], axis=1)
    o = jnp.dot(a, wl_ref[...], preferred_element_type=jnp.float32)
    o = o + bl_ref[...]
    o = o + jnp.dot(h, wr_ref[...], preferred_element_type=jnp.float32)
    o = jnp.maximum(o, 0.0)
    o0_ref[...] = o[:, :HALF]
    o1_ref[...] = o[:, HALF:]


def _combine(a0, a1, deg, h0, h1, wl, bl, wr, rows=1000):
    n = h0.shape[0]
    return pl.pallas_call(
        _combine_block,
        grid=(n // rows,),
        in_specs=[pl.BlockSpec((rows, HALF), lambda i: (i, 0)),
                  pl.BlockSpec((rows, HALF), lambda i: (i, 0)),
                  pl.BlockSpec((rows, 1), lambda i: (i, 0)),
                  pl.BlockSpec((rows, HALF), lambda i: (i, 0)),
                  pl.BlockSpec((rows, HALF), lambda i: (i, 0)),
                  pl.BlockSpec((DE, DH), lambda i: (0, 0)),
                  pl.BlockSpec((1, DH), lambda i: (0, 0)),
                  pl.BlockSpec((DE, DH), lambda i: (0, 0))],
        out_specs=[pl.BlockSpec((rows, HALF), lambda i: (i, 0)),
                   pl.BlockSpec((rows, HALF), lambda i: (i, 0))],
        out_shape=[jax.ShapeDtypeStruct((n, HALF), jnp.float32)] * 2,
    )(a0, a1, deg.reshape(n, 1) if deg.shape[0] == n else deg[:n].reshape(n, 1),
      h0, h1, wl, bl.reshape(1, DH), wr)


def _pred_block(au0, au1, du, ue0, ue1, aj0, aj1, dj, je0, je1,
                vp_ref, c_ref, o_ref):
    vp = vp_ref[...]
    ru = 1.0 / jnp.maximum(du[...], 1.0)
    rj = 1.0 / jnp.maximum(dj[...], 1.0)
    s = jnp.sum(au0[...] * vp[0:1] + au1[...] * vp[1:2], axis=1, keepdims=True) * ru
    s = s + jnp.sum(ue0[...] * vp[2:3] + ue1[...] * vp[3:4], axis=1, keepdims=True)
    s = s + jnp.sum(aj0[...] * vp[4:5] + aj1[...] * vp[5:6], axis=1, keepdims=True) * rj
    s = s + jnp.sum(je0[...] * vp[6:7] + je1[...] * vp[7:8], axis=1, keepdims=True)
    o_ref[...] = s + c_ref[...]


def _pred(au0, au1, du, ue0, ue1, aj0, aj1, dj, je0, je1, vpack, c):
    n = au0.shape[0]
    half_spec = pl.BlockSpec((n, HALF), lambda: (0, 0))
    one_spec = pl.BlockSpec((n, 1), lambda: (0, 0))
    return pl.pallas_call(
        _pred_block,
        in_specs=[half_spec, half_spec, one_spec, half_spec, half_spec,
                  half_spec, half_spec, one_spec, half_spec, half_spec,
                  pl.BlockSpec((8, HALF), lambda: (0, 0)),
                  pl.BlockSpec((1, 1), lambda: (0, 0))],
        out_specs=pl.BlockSpec((n, 1), lambda: (0, 0)),
        out_shape=jax.ShapeDtypeStruct((n, 1), jnp.float32),
    )(au0, au1, du.reshape(n, 1), ue0, ue1, aj0, aj1, dj.reshape(n, 1),
      je0, je1, vpack, c.reshape(1, 1))


def _segsum_body(with_deg, *refs):
    if with_deg:
        (t0h, t1h, idxg_h, idxs_h, idxd_h, zrow_h, zdeg_h,
         o0, o1, dj_o, du_o,
         ig0, ig1, is0, is1, r0, r1, id0, id1, ones_v, dv_v,
         acc_sh, deg_sh, sg0, sg1, ss0, ss1) = refs
        ids = (id0, id1)
    else:
        (t0h, t1h, idxg_h, idxs_h, zrow_h,
         o0, o1,
         ig0, ig1, is0, is1, r0, r1, acc_sh, sg0, sg1, ss0, ss1) = refs
    igs = (ig0, ig1)
    iss = (is0, is1)
    rows = (r0, r1)
    sgs = (sg0, sg1)
    sss = (ss0, ss1)
    c = lax.axis_index("c")
    t = lax.axis_index("s")

    for i in range(4):
        pltpu.sync_copy(zrow_h, acc_sh.at[pl.ds(t * RPT + i * WB, WB)])
    if with_deg:
        pltpu.sync_copy(zdeg_h, dv_v)
        pltpu.sync_copy(dv_v, deg_sh.at[pl.ds(t * RPT, RPT)])
        for i in range(W // 16):
            ones_v[pl.ds(i * 16, 16)] = jnp.full((16,), 1.0, jnp.float32)
    plsc.subcore_barrier()

    tbase = t * (NWIN * KS2)

    def load_and_fire(w, b):
        rowbase = tbase + w * KS2
        pltpu.sync_copy(idxg_h.at[pl.ds(rowbase, KS2)], igs[b])
        pltpu.sync_copy(idxs_h.at[pl.ds(rowbase, KS2)], iss[b])
        if with_deg:
            pltpu.sync_copy(idxd_h.at[pl.ds(c * EROWS + rowbase, KS2)], ids[b])

        def fire(tab):
            def go():
                for j in range(KS2):
                    pltpu.async_copy(tab.at[igs[b].at[j]],
                                     rows[b].at[pl.ds(j * W, W)], sgs[b])
            return go
        pl.when(c == 0)(fire(t0h))
        pl.when(c == 1)(fire(t1h))

    def drain_gathers(b):
        pltpu.make_async_copy(t0h.at[pl.ds(0, KS2 * W)], rows[b],
                              sgs[b]).wait()

    def fire_scatters(b):
        for j in range(KS2):
            pltpu.async_copy(rows[b].at[pl.ds(j * W, W)],
                             acc_sh.at[iss[b].at[j]], sss[b], add=True)
        if with_deg:
            for j in range(KS2):
                pltpu.sync_copy(ones_v, deg_sh.at[ids[b].at[j]], add=True)

    def drain_scatters(b):
        pltpu.make_async_copy(t0h.at[pl.ds(0, KS2 * W)], rows[b],
                              sss[b]).wait()

    load_and_fire(0, 0)
    load_and_fire(1, 1)

    def outer(g, carry):
        for b in range(2):
            drain_gathers(b)
            fire_scatters(b)

            @pl.when(g < NGRP - 1)
            def _():
                drain_scatters(b)
                load_and_fire(g * 2 + b + 2, b)
        return carry

    lax.fori_loop(0, NGRP, outer, 0)
    drain_scatters(0)
    drain_scatters(1)
    plsc.subcore_barrier()

    def wb(dst):
        def go():
            for i in range(4):
                r0 = t * RPT + i * WB
                pltpu.sync_copy(acc_sh.at[pl.ds(r0, WB)], dst.at[pl.ds(r0, WB)])
        return go
    pl.when(c == 0)(wb(o0))
    pl.when(c == 1)(wb(o1))
    if with_deg:
        def wbd(dst):
            def go():
                pltpu.sync_copy(deg_sh.at[pl.ds(t * RPT, RPT)], dv_v)
                pltpu.sync_copy(dv_v, dst.at[pl.ds(t * RPT, RPT)])
            return go
        pl.when(c == 0)(wbd(dj_o))
        pl.when(c == 1)(wbd(du_o))


def _make_segsum(with_deg):
    out_type = [jax.ShapeDtypeStruct((NPAD, HALF), jnp.float32)] * 2
    ring = [pltpu.VMEM((KS2, W), jnp.int32)] * 4 \
        + [pltpu.VMEM((KS2 * W, HALF), jnp.float32)] * 2
    sems = [pltpu.SemaphoreType.DMA] * 4
    if with_deg:
        out_type = out_type + [jax.ShapeDtypeStruct((NPAD,), jnp.float32)] * 2
        scratch = ring + [pltpu.VMEM((KS2, W), jnp.int32)] * 2 \
            + [pltpu.VMEM((W,), jnp.float32),
               pltpu.VMEM((RPT,), jnp.float32),
               pltpu.VMEM_SHARED((NPAD, HALF), jnp.float32),
               pltpu.VMEM_SHARED((NPAD,), jnp.float32)] + sems
    else:
        scratch = ring + [pltpu.VMEM_SHARED((NPAD, HALF), jnp.float32)] + sems
    return pl.kernel(functools.partial(_segsum_body, with_deg),
                     out_type=out_type, mesh=_MESH, scratch_types=scratch,
                     compiler_params=pltpu.CompilerParams(
                         use_tc_tiling_on_sc=False))


_segsum = _make_segsum(False)
_segsum_deg = _make_segsum(True)


def _gathersel_body(*refs):
    (au0h, au1h, nu0h, nu1h, aj0h, aj1h, nj0h, nj1h, du_h, dj_h, ui_h, ji_h,
     oau0, oau1, onu0, onu1, oaj0, oaj1, onj0, onj1, odu, odj,
     iu_v, ij_v, b0, b1, b2, b3, b4, b5, b6, b7, du_v, dj_v, sem) = refs
    c = lax.axis_index("c")
    t = lax.axis_index("s")
    base = (t * NSC + c) * W
    pltpu.sync_copy(ui_h.at[pl.ds(base, W)], iu_v)
    pltpu.sync_copy(ji_h.at[pl.ds(base, W)], ij_v)
    rows = [(au0h, b0, oau0, iu_v), (au1h, b1, oau1, iu_v),
            (nu0h, b2, onu0, iu_v), (nu1h, b3, onu1, iu_v),
            (aj0h, b4, oaj0, ij_v), (aj1h, b5, oaj1, ij_v),
            (nj0h, b6, onj0, ij_v), (nj1h, b7, onj1, ij_v)]
    descs = [pltpu.async_copy(tab.at[iv], buf, sem)
             for tab, buf, _, iv in rows]
    descs.append(pltpu.async_copy(du_h.at[iu_v], du_v, sem))
    descs.append(pltpu.async_copy(dj_h.at[ij_v], dj_v, sem))
    for d in descs:
        d.wait()
    for _, buf, out, _ in rows:
        pltpu.sync_copy(buf, out.at[pl.ds(base, W)])
    pltpu.sync_copy(du_v, odu.at[pl.ds(base, W)])
    pltpu.sync_copy(dj_v, odj.at[pl.ds(base, W)])


_gathersel = pl.kernel(
    _gathersel_body,
    out_type=[jax.ShapeDtypeStruct((BSEL, HALF), jnp.float32)] * 8
             + [jax.ShapeDtypeStruct((BSEL,), jnp.float32)] * 2,
    mesh=_MESH,
    scratch_types=[pltpu.VMEM((W,), jnp.int32)] * 2
                  + [pltpu.VMEM((W, HALF), jnp.float32)] * 8
                  + [pltpu.VMEM((W,), jnp.float32)] * 2
                  + [pltpu.SemaphoreType.DMA],
    compiler_params=pltpu.CompilerParams(use_tc_tiling_on_sc=False))


def kernel(x_user, x_job, edge_index, user_indices, job_indices,
           W_emb_user, b_emb_user, W_emb_job, b_emb_job,
           Wl0_uj, bl0_uj, Wr0_uj, Wl0_ju, bl0_ju, Wr0_ju,
           Wl1_uj, bl1_uj, Wr1_uj, Wl1_ju, bl1_ju, Wr1_ju,
           W_pred, b_pred):
    src = edge_index[0]
    dst = edge_index[1]

    hu0, hu1 = _embed(x_user, W_emb_user, b_emb_user)
    hj0, hj1 = _embed(x_job, W_emb_job, b_emb_job)

    pad_n = EPAD - EDGES
    ar = jnp.arange(pad_n, dtype=jnp.int32)
    pad_g = ar % 128
    pad_s = NJ + ar % (NPAD - NJ)
    src_g = jnp.concatenate([src, pad_g]).reshape(EROWS, W)
    dst_s = jnp.concatenate([dst, pad_s]).reshape(EROWS, W)
    dst_g = jnp.concatenate([dst, pad_g]).reshape(EROWS, W)
    src_s = jnp.concatenate([src, pad_s]).reshape(EROWS, W)
    idxd = jnp.concatenate([dst_s, src_s], axis=0)
    zrow = jnp.zeros((WB, HALF), jnp.float32)
    zdeg = jnp.zeros((RPT,), jnp.float32)

    aj0, aj1, deg_j, deg_u = _segsum_deg(hu0, hu1, src_g, dst_s, idxd,
                                         zrow, zdeg)
    au0, au1 = _segsum(hj0, hj1, dst_g, src_s, zrow)
    nj0, nj1 = _combine(aj0, aj1, deg_j, hj0, hj1, Wl0_uj, bl0_uj, Wr0_uj)
    nu0, nu1 = _combine(au0, au1, deg_u, hu0, hu1, Wl0_ju, bl0_ju, Wr0_ju)

    aj0, aj1 = _segsum(nu0, nu1, src_g, dst_s, zrow)
    au0, au1 = _segsum(nj0, nj1, dst_g, src_s, zrow)

    wp_u = W_pred[:DH, 0]
    wp_j = W_pred[DH:, 0]
    v1 = Wl1_ju @ wp_u
    v2 = Wr1_ju @ wp_u
    v3 = Wl1_uj @ wp_j
    v4 = Wr1_uj @ wp_j
    c = bl1_ju @ wp_u + bl1_uj @ wp_j + b_pred[0]
    vpack = jnp.stack([v1[:HALF], v1[HALF:], v2[:HALF], v2[HALF:],
                       v3[:HALF], v3[HALF:], v4[:HALF], v4[HALF:]], axis=0)

    (au0s, au1s, ue0, ue1, aj0s, aj1s, je0, je1, dus, djs) = _gathersel(
        au0, au1, nu0, nu1, aj0, aj1, nj0, nj1, deg_u, deg_j,
        user_indices, job_indices)

    preds = _pred(au0s, au1s, dus, ue0, ue1, aj0s, aj1s, djs, je0, je1,
                  vpack, c)
    return preds[:, 0]


# speedup vs baseline: 1.8547x; 1.2514x over previous
import functools

import jax
import jax.numpy as jnp
from jax import lax
from jax.experimental import pallas as pl
from jax.experimental.pallas import tpu as pltpu
from jax.experimental.pallas import tpu_sc as plsc

NU = 50000
NJ = 50000
EDGES = 800000
DF = 128
DE = 64
DH = 64
BSEL = 4096
HALF = 32

NSC = 2
NTILE = 16
W = 128
KSUB = 4
SUPS = 98
KS2 = 2
NWIN = SUPS * KSUB // KS2
NGRP = NWIN // 2
TILE_E = SUPS * KSUB * W
EPAD = NTILE * TILE_E
EROWS = EPAD // W
NPAD = 50176
RPT = NPAD // NTILE
WB = RPT // 4

_MESH = plsc.VectorSubcoreMesh(core_axis_name="c", subcore_axis_name="s",
                               num_cores=NSC, num_subcores=NTILE)


def _embed_block(x_ref, w_ref, b_ref, h0_ref, h1_ref):
    h = jnp.dot(x_ref[...], w_ref[...], preferred_element_type=jnp.float32)
    h = h + b_ref[...]
    h0_ref[...] = h[:, :HALF]
    h1_ref[...] = h[:, HALF:]


def _embed(x, w, b, rows=1000):
    n = x.shape[0]
    return pl.pallas_call(
        _embed_block,
        grid=(n // rows,),
        in_specs=[pl.BlockSpec((rows, DF), lambda i: (i, 0)),
                  pl.BlockSpec((DF, DE), lambda i: (0, 0)),
                  pl.BlockSpec((1, DE), lambda i: (0, 0))],
        out_specs=[pl.BlockSpec((rows, HALF), lambda i: (i, 0)),
                   pl.BlockSpec((rows, HALF), lambda i: (i, 0))],
        out_shape=[jax.ShapeDtypeStruct((n, HALF), jnp.float32)] * 2,
    )(x, w, b.reshape(1, DE))


def _combine_block(a0_ref, a1_ref, d_ref, h0_ref, h1_ref,
                   wl_ref, bl_ref, wr_ref, o0_ref, o1_ref):
    r = 1.0 / jnp.maximum(d_ref[...], 1.0)
    a = jnp.concatenate([a0_ref[...] * r, a1_ref[...] * r], axis=1)
    h = jnp.concatenate([h0_ref[...], h1_ref[...]], axis=1)
    o = jnp.dot(a, wl_ref[...], preferred_element_type=jnp.float32)
    o = o + bl_ref[...]
    o = o + jnp.dot(h, wr_ref[...], preferred_element_type=jnp.float32)
    o = jnp.maximum(o, 0.0)
    o0_ref[...] = o[:, :HALF]
    o1_ref[...] = o[:, HALF:]


def _combine(a0, a1, deg, h0, h1, wl, bl, wr, rows=1000):
    n = h0.shape[0]
    return pl.pallas_call(
        _combine_block,
        grid=(n // rows,),
        in_specs=[pl.BlockSpec((rows, HALF), lambda i: (i, 0)),
                  pl.BlockSpec((rows, HALF), lambda i: (i, 0)),
                  pl.BlockSpec((rows, 1), lambda i: (i, 0)),
                  pl.BlockSpec((rows, HALF), lambda i: (i, 0)),
                  pl.BlockSpec((rows, HALF), lambda i: (i, 0)),
                  pl.BlockSpec((DE, DH), lambda i: (0, 0)),
                  pl.BlockSpec((1, DH), lambda i: (0, 0)),
                  pl.BlockSpec((DE, DH), lambda i: (0, 0))],
        out_specs=[pl.BlockSpec((rows, HALF), lambda i: (i, 0)),
                   pl.BlockSpec((rows, HALF), lambda i: (i, 0))],
        out_shape=[jax.ShapeDtypeStruct((n, HALF), jnp.float32)] * 2,
    )(a0, a1, deg.reshape(n, 1) if deg.shape[0] == n else deg[:n].reshape(n, 1),
      h0, h1, wl, bl.reshape(1, DH), wr)


def _pred_block(au0, au1, du, ue0, ue1, aj0, aj1, dj, je0, je1,
                vp_ref, c_ref, o_ref):
    vp = vp_ref[...]
    ru = 1.0 / jnp.maximum(du[...], 1.0)
    rj = 1.0 / jnp.maximum(dj[...], 1.0)
    s = jnp.sum(au0[...] * vp[0:1] + au1[...] * vp[1:2], axis=1, keepdims=True) * ru
    s = s + jnp.sum(ue0[...] * vp[2:3] + ue1[...] * vp[3:4], axis=1, keepdims=True)
    s = s + jnp.sum(aj0[...] * vp[4:5] + aj1[...] * vp[5:6], axis=1, keepdims=True) * rj
    s = s + jnp.sum(je0[...] * vp[6:7] + je1[...] * vp[7:8], axis=1, keepdims=True)
    o_ref[...] = s + c_ref[...]


def _pred(au0, au1, du, ue0, ue1, aj0, aj1, dj, je0, je1, vpack, c):
    n = au0.shape[0]
    half_spec = pl.BlockSpec((n, HALF), lambda: (0, 0))
    one_spec = pl.BlockSpec((n, 1), lambda: (0, 0))
    return pl.pallas_call(
        _pred_block,
        in_specs=[half_spec, half_spec, one_spec, half_spec, half_spec,
                  half_spec, half_spec, one_spec, half_spec, half_spec,
                  pl.BlockSpec((8, HALF), lambda: (0, 0)),
                  pl.BlockSpec((1, 1), lambda: (0, 0))],
        out_specs=pl.BlockSpec((n, 1), lambda: (0, 0)),
        out_shape=jax.ShapeDtypeStruct((n, 1), jnp.float32),
    )(au0, au1, du.reshape(n, 1), ue0, ue1, aj0, aj1, dj.reshape(n, 1),
      je0, je1, vpack, c.reshape(1, 1))


def _segsum_body(with_deg, *refs):
    if with_deg:
        (t0h, t1h, idxgs_h, idxd_h, zrow_h, zdeg_h,
         o0, o1, dj_o, du_o,
         i0, i1, r0, r1, id0, id1, ones_v, dv_v,
         acc_sh, deg_sh, sg0, sg1, ss0, ss1) = refs
        ids = (id0, id1)
    else:
        (t0h, t1h, idxgs_h, zrow_h,
         o0, o1,
         i0, i1, r0, r1, acc_sh, sg0, sg1, ss0, ss1) = refs
    ixs = (i0, i1)
    rows = (r0, r1)
    sgs = (sg0, sg1)
    sss = (ss0, ss1)
    c = lax.axis_index("c")
    t = lax.axis_index("s")

    for i in range(4):
        pltpu.sync_copy(zrow_h, acc_sh.at[pl.ds(t * RPT + i * WB, WB)])
    if with_deg:
        pltpu.sync_copy(zdeg_h, dv_v)
        pltpu.sync_copy(dv_v, deg_sh.at[pl.ds(t * RPT, RPT)])
        for i in range(W // 16):
            ones_v[pl.ds(i * 16, 16)] = jnp.full((16,), 1.0, jnp.float32)
    plsc.subcore_barrier()

    tbase = t * NWIN

    def load_and_fire(w, b):
        gw = tbase + w
        pltpu.sync_copy(idxgs_h.at[pl.ds(gw * 2 * KS2, 2 * KS2)], ixs[b])
        if with_deg:
            pltpu.sync_copy(
                idxd_h.at[pl.ds(c * EROWS + gw * KS2, KS2)], ids[b])

        def fire(tab):
            def go():
                for j in range(KS2):
                    pltpu.async_copy(tab.at[ixs[b].at[j]],
                                     rows[b].at[pl.ds(j * W, W)], sgs[b])
            return go
        pl.when(c == 0)(fire(t0h))
        pl.when(c == 1)(fire(t1h))

    def drain_gathers(b):
        pltpu.make_async_copy(t0h.at[pl.ds(0, KS2 * W)], rows[b],
                              sgs[b]).wait()

    def fire_scatters(b):
        for j in range(KS2):
            pltpu.async_copy(rows[b].at[pl.ds(j * W, W)],
                             acc_sh.at[ixs[b].at[KS2 + j]], sss[b], add=True)
        if with_deg:
            for j in range(KS2):
                pltpu.sync_copy(ones_v, deg_sh.at[ids[b].at[j]], add=True)

    def drain_scatters(b):
        pltpu.make_async_copy(t0h.at[pl.ds(0, KS2 * W)], rows[b],
                              sss[b]).wait()

    load_and_fire(0, 0)
    load_and_fire(1, 1)

    def outer(g, carry):
        for b in range(2):
            drain_gathers(b)
            fire_scatters(b)

            @pl.when(g < NGRP - 1)
            def _():
                drain_scatters(b)
                load_and_fire(g * 2 + b + 2, b)
        return carry

    lax.fori_loop(0, NGRP, outer, 0)
    drain_scatters(0)
    drain_scatters(1)
    plsc.subcore_barrier()

    def wb(dst):
        def go():
            for i in range(4):
                r0 = t * RPT + i * WB
                pltpu.sync_copy(acc_sh.at[pl.ds(r0, WB)], dst.at[pl.ds(r0, WB)])
        return go
    pl.when(c == 0)(wb(o0))
    pl.when(c == 1)(wb(o1))
    if with_deg:
        def wbd(dst):
            def go():
                pltpu.sync_copy(deg_sh.at[pl.ds(t * RPT, RPT)], dv_v)
                pltpu.sync_copy(dv_v, dst.at[pl.ds(t * RPT, RPT)])
            return go
        pl.when(c == 0)(wbd(dj_o))
        pl.when(c == 1)(wbd(du_o))


def _make_segsum(with_deg):
    out_type = [jax.ShapeDtypeStruct((NPAD, HALF), jnp.float32)] * 2
    ring = [pltpu.VMEM((2 * KS2, W), jnp.int32)] * 2 \
        + [pltpu.VMEM((KS2 * W, HALF), jnp.float32)] * 2
    sems = [pltpu.SemaphoreType.DMA] * 4
    if with_deg:
        out_type = out_type + [jax.ShapeDtypeStruct((NPAD,), jnp.float32)] * 2
        scratch = ring + [pltpu.VMEM((KS2, W), jnp.int32)] * 2 \
            + [pltpu.VMEM((W,), jnp.float32),
               pltpu.VMEM((RPT,), jnp.float32),
               pltpu.VMEM_SHARED((NPAD, HALF), jnp.float32),
               pltpu.VMEM_SHARED((NPAD,), jnp.float32)] + sems
    else:
        scratch = ring + [pltpu.VMEM_SHARED((NPAD, HALF), jnp.float32)] + sems
    return pl.kernel(functools.partial(_segsum_body, with_deg),
                     out_type=out_type, mesh=_MESH, scratch_types=scratch,
                     compiler_params=pltpu.CompilerParams(
                         use_tc_tiling_on_sc=False))


_segsum = _make_segsum(False)
_segsum_deg = _make_segsum(True)


def _gathersel_body(*refs):
    (au0h, au1h, nu0h, nu1h, aj0h, aj1h, nj0h, nj1h, du_h, dj_h, ui_h, ji_h,
     oau0, oau1, onu0, onu1, oaj0, oaj1, onj0, onj1, odu, odj,
     iu_v, ij_v, b0, b1, b2, b3, b4, b5, b6, b7, du_v, dj_v, sem) = refs
    c = lax.axis_index("c")
    t = lax.axis_index("s")
    base = (t * NSC + c) * W
    pltpu.sync_copy(ui_h.at[pl.ds(base, W)], iu_v)
    pltpu.sync_copy(ji_h.at[pl.ds(base, W)], ij_v)
    rows = [(au0h, b0, oau0, iu_v), (au1h, b1, oau1, iu_v),
            (nu0h, b2, onu0, iu_v), (nu1h, b3, onu1, iu_v),
            (aj0h, b4, oaj0, ij_v), (aj1h, b5, oaj1, ij_v),
            (nj0h, b6, onj0, ij_v), (nj1h, b7, onj1, ij_v)]
    descs = [pltpu.async_copy(tab.at[iv], buf, sem)
             for tab, buf, _, iv in rows]
    descs.append(pltpu.async_copy(du_h.at[iu_v], du_v, sem))
    descs.append(pltpu.async_copy(dj_h.at[ij_v], dj_v, sem))
    for d in descs:
        d.wait()
    for _, buf, out, _ in rows:
        pltpu.sync_copy(buf, out.at[pl.ds(base, W)])
    pltpu.sync_copy(du_v, odu.at[pl.ds(base, W)])
    pltpu.sync_copy(dj_v, odj.at[pl.ds(base, W)])


_gathersel = pl.kernel(
    _gathersel_body,
    out_type=[jax.ShapeDtypeStruct((BSEL, HALF), jnp.float32)] * 8
             + [jax.ShapeDtypeStruct((BSEL,), jnp.float32)] * 2,
    mesh=_MESH,
    scratch_types=[pltpu.VMEM((W,), jnp.int32)] * 2
                  + [pltpu.VMEM((W, HALF), jnp.float32)] * 8
                  + [pltpu.VMEM((W,), jnp.float32)] * 2
                  + [pltpu.SemaphoreType.DMA],
    compiler_params=pltpu.CompilerParams(use_tc_tiling_on_sc=False))


def kernel(x_user, x_job, edge_index, user_indices, job_indices,
           W_emb_user, b_emb_user, W_emb_job, b_emb_job,
           Wl0_uj, bl0_uj, Wr0_uj, Wl0_ju, bl0_ju, Wr0_ju,
           Wl1_uj, bl1_uj, Wr1_uj, Wl1_ju, bl1_ju, Wr1_ju,
           W_pred, b_pred):
    src = edge_index[0]
    dst = edge_index[1]

    hu0, hu1 = _embed(x_user, W_emb_user, b_emb_user)
    hj0, hj1 = _embed(x_job, W_emb_job, b_emb_job)

    pad_n = EPAD - EDGES
    ar = jnp.arange(pad_n, dtype=jnp.int32)
    pad_g = ar % 128
    pad_s = NJ + ar % (NPAD - NJ)
    src_g = jnp.concatenate([src, pad_g]).reshape(EROWS, W)
    dst_s = jnp.concatenate([dst, pad_s]).reshape(EROWS, W)
    dst_g = jnp.concatenate([dst, pad_g]).reshape(EROWS, W)
    src_s = jnp.concatenate([src, pad_s]).reshape(EROWS, W)
    idxd = jnp.concatenate([dst_s, src_s], axis=0)
    zrow = jnp.zeros((WB, HALF), jnp.float32)
    zdeg = jnp.zeros((RPT,), jnp.float32)

    def pack(g_rows, s_rows):
        nwt = EROWS // KS2
        both = jnp.concatenate([g_rows.reshape(nwt, KS2, W),
                                s_rows.reshape(nwt, KS2, W)], axis=1)
        return both.reshape(nwt * 2 * KS2, W)

    idx_fw = pack(src_g, dst_s)
    idx_bw = pack(dst_g, src_s)

    aj0, aj1, deg_j, deg_u = _segsum_deg(hu0, hu1, idx_fw, idxd,
                                         zrow, zdeg)
    au0, au1 = _segsum(hj0, hj1, idx_bw, zrow)
    nj0, nj1 = _combine(aj0, aj1, deg_j, hj0, hj1, Wl0_uj, bl0_uj, Wr0_uj)
    nu0, nu1 = _combine(au0, au1, deg_u, hu0, hu1, Wl0_ju, bl0_ju, Wr0_ju)

    aj0, aj1 = _segsum(nu0, nu1, idx_fw, zrow)
    au0, au1 = _segsum(nj0, nj1, idx_bw, zrow)

    wp_u = W_pred[:DH, 0]
    wp_j = W_pred[DH:, 0]
    v1 = Wl1_ju @ wp_u
    v2 = Wr1_ju @ wp_u
    v3 = Wl1_uj @ wp_j
    v4 = Wr1_uj @ wp_j
    c = bl1_ju @ wp_u + bl1_uj @ wp_j + b_pred[0]
    vpack = jnp.stack([v1[:HALF], v1[HALF:], v2[:HALF], v2[HALF:],
                       v3[:HALF], v3[HALF:], v4[:HALF], v4[HALF:]], axis=0)

    (au0s, au1s, ue0, ue1, aj0s, aj1s, je0, je1, dus, djs) = _gathersel(
        au0, au1, nu0, nu1, aj0, aj1, nj0, nj1, deg_u, deg_j,
        user_indices, job_indices)

    preds = _pred(au0s, au1s, dus, ue0, ue1, aj0s, aj1s, djs, je0, je1,
                  vpack, c)
    return preds[:, 0]


# speedup vs baseline: 2.3859x; 1.2864x over previous
import functools

import jax
import jax.numpy as jnp
from jax import lax
from jax.experimental import pallas as pl
from jax.experimental.pallas import tpu as pltpu
from jax.experimental.pallas import tpu_sc as plsc

NU = 50000
NJ = 50000
EDGES = 800000
DF = 128
DE = 64
DH = 64
BSEL = 4096
HALF = 32

NSC = 2
NTILE = 16
W = 128
KSUB = 4
SUPS = 98
KS2 = 2
NWIN = SUPS * KSUB // KS2
NGRP = NWIN // 2
TILE_E = SUPS * KSUB * W
EPAD = NTILE * TILE_E
EROWS = EPAD // W
NPAD = 50176
RPT = NPAD // NTILE
WB = RPT // 4

_MESH = plsc.VectorSubcoreMesh(core_axis_name="c", subcore_axis_name="s",
                               num_cores=NSC, num_subcores=NTILE)


def _embed_block(x_ref, w_ref, b_ref, h0_ref, h1_ref):
    h = jnp.dot(x_ref[...], w_ref[...], preferred_element_type=jnp.float32)
    h = h + b_ref[...]
    h0_ref[...] = h[:, :HALF]
    h1_ref[...] = h[:, HALF:]


def _embed(x, w, b, rows=1000):
    n = x.shape[0]
    return pl.pallas_call(
        _embed_block,
        grid=(n // rows,),
        in_specs=[pl.BlockSpec((rows, DF), lambda i: (i, 0)),
                  pl.BlockSpec((DF, DE), lambda i: (0, 0)),
                  pl.BlockSpec((1, DE), lambda i: (0, 0))],
        out_specs=[pl.BlockSpec((rows, HALF), lambda i: (i, 0)),
                   pl.BlockSpec((rows, HALF), lambda i: (i, 0))],
        out_shape=[jax.ShapeDtypeStruct((n, HALF), jnp.float32)] * 2,
    )(x, w, b.reshape(1, DE))


def _combine_block(a0_ref, a1_ref, d_ref, h0_ref, h1_ref,
                   wl_ref, bl_ref, wr_ref, o0_ref, o1_ref):
    r = 1.0 / jnp.maximum(d_ref[...], 1.0)
    a = jnp.concatenate([a0_ref[...] * r, a1_ref[...] * r], axis=1)
    h = jnp.concatenate([h0_ref[...], h1_ref[...]], axis=1)
    o = jnp.dot(a, wl_ref[...], preferred_element_type=jnp.float32)
    o = o + bl_ref[...]
    o = o + jnp.dot(h, wr_ref[...], preferred_element_type=jnp.float32)
    o = jnp.maximum(o, 0.0)
    o0_ref[...] = o[:, :HALF]
    o1_ref[...] = o[:, HALF:]


def _combine(a0, a1, deg, h0, h1, wl, bl, wr, rows=1000):
    n = h0.shape[0]
    return pl.pallas_call(
        _combine_block,
        grid=(n // rows,),
        in_specs=[pl.BlockSpec((rows, HALF), lambda i: (i, 0)),
                  pl.BlockSpec((rows, HALF), lambda i: (i, 0)),
                  pl.BlockSpec((rows, 1), lambda i: (i, 0)),
                  pl.BlockSpec((rows, HALF), lambda i: (i, 0)),
                  pl.BlockSpec((rows, HALF), lambda i: (i, 0)),
                  pl.BlockSpec((DE, DH), lambda i: (0, 0)),
                  pl.BlockSpec((1, DH), lambda i: (0, 0)),
                  pl.BlockSpec((DE, DH), lambda i: (0, 0))],
        out_specs=[pl.BlockSpec((rows, HALF), lambda i: (i, 0)),
                   pl.BlockSpec((rows, HALF), lambda i: (i, 0))],
        out_shape=[jax.ShapeDtypeStruct((n, HALF), jnp.float32)] * 2,
    )(a0, a1, deg.reshape(n, 1) if deg.shape[0] == n else deg[:n].reshape(n, 1),
      h0, h1, wl, bl.reshape(1, DH), wr)


def _pred_block(au0, au1, du, ue0, ue1, aj0, aj1, dj, je0, je1,
                vp_ref, c_ref, o_ref):
    vp = vp_ref[...]
    ru = 1.0 / jnp.maximum(du[...], 1.0)
    rj = 1.0 / jnp.maximum(dj[...], 1.0)
    s = jnp.sum(au0[...] * vp[0:1] + au1[...] * vp[1:2], axis=1, keepdims=True) * ru
    s = s + jnp.sum(ue0[...] * vp[2:3] + ue1[...] * vp[3:4], axis=1, keepdims=True)
    s = s + jnp.sum(aj0[...] * vp[4:5] + aj1[...] * vp[5:6], axis=1, keepdims=True) * rj
    s = s + jnp.sum(je0[...] * vp[6:7] + je1[...] * vp[7:8], axis=1, keepdims=True)
    o_ref[...] = s + c_ref[...]


def _pred(au0, au1, du, ue0, ue1, aj0, aj1, dj, je0, je1, vpack, c):
    n = au0.shape[0]
    half_spec = pl.BlockSpec((n, HALF), lambda: (0, 0))
    one_spec = pl.BlockSpec((n, 1), lambda: (0, 0))
    return pl.pallas_call(
        _pred_block,
        in_specs=[half_spec, half_spec, one_spec, half_spec, half_spec,
                  half_spec, half_spec, one_spec, half_spec, half_spec,
                  pl.BlockSpec((8, HALF), lambda: (0, 0)),
                  pl.BlockSpec((1, 1), lambda: (0, 0))],
        out_specs=pl.BlockSpec((n, 1), lambda: (0, 0)),
        out_shape=jax.ShapeDtypeStruct((n, 1), jnp.float32),
    )(au0, au1, du.reshape(n, 1), ue0, ue1, aj0, aj1, dj.reshape(n, 1),
      je0, je1, vpack, c.reshape(1, 1))


def _segsum_body(with_deg, *refs):
    if with_deg:
        (t0h, t1h, idxgs_h, idxd_h, zrow_h, zdeg_h,
         o0, o1, dj_o, du_o,
         ix00, ix01, ix10, ix11, r0, r1,
         id00, id01, id10, id11, ones_v, dv_v,
         acc_sh, deg_sh, sg0, sg1, ss0, ss1, si0, si1) = refs
        ids = ((id00, id01), (id10, id11))
    else:
        (t0h, t1h, idxgs_h, zrow_h,
         o0, o1,
         ix00, ix01, ix10, ix11, r0, r1,
         acc_sh, sg0, sg1, ss0, ss1, si0, si1) = refs
    ixs = ((ix00, ix01), (ix10, ix11))
    rows = (r0, r1)
    sgs = (sg0, sg1)
    sss = (ss0, ss1)
    sis = (si0, si1)
    c = lax.axis_index("c")
    t = lax.axis_index("s")

    for i in range(4):
        pltpu.sync_copy(zrow_h, acc_sh.at[pl.ds(t * RPT + i * WB, WB)])
    if with_deg:
        pltpu.sync_copy(zdeg_h, dv_v)
        pltpu.sync_copy(dv_v, deg_sh.at[pl.ds(t * RPT, RPT)])
        for i in range(W // 16):
            ones_v[pl.ds(i * 16, 16)] = jnp.full((16,), 1.0, jnp.float32)
    plsc.subcore_barrier()

    tbase = t * NWIN

    def load_idx(w, b, p):
        gw = tbase + w
        pltpu.async_copy(idxgs_h.at[pl.ds(gw * 2 * KS2, 2 * KS2)],
                         ixs[b][p], sis[b])
        if with_deg:
            pltpu.async_copy(idxd_h.at[pl.ds(c * EROWS + gw * KS2, KS2)],
                             ids[b][p], sis[b])

    def drain_idx(b, p):
        pltpu.make_async_copy(idxgs_h.at[pl.ds(0, 2 * KS2)], ixs[b][p],
                              sis[b]).wait()
        if with_deg:
            pltpu.make_async_copy(idxd_h.at[pl.ds(0, KS2)], ids[b][p],
                                  sis[b]).wait()

    def fire_gathers(b, p):
        def fire(tab):
            def go():
                for j in range(KS2):
                    pltpu.async_copy(tab.at[ixs[b][p].at[j]],
                                     rows[b].at[pl.ds(j * W, W)], sgs[b])
            return go
        pl.when(c == 0)(fire(t0h))
        pl.when(c == 1)(fire(t1h))

    def drain_gathers(b):
        pltpu.make_async_copy(t0h.at[pl.ds(0, KS2 * W)], rows[b],
                              sgs[b]).wait()

    def fire_scatters(b, p):
        for j in range(KS2):
            pltpu.async_copy(rows[b].at[pl.ds(j * W, W)],
                             acc_sh.at[ixs[b][p].at[KS2 + j]], sss[b],
                             add=True)
        if with_deg:
            for j in range(KS2):
                pltpu.sync_copy(ones_v, deg_sh.at[ids[b][p].at[j]],
                                add=True)

    def drain_scatters(b):
        pltpu.make_async_copy(t0h.at[pl.ds(0, KS2 * W)], rows[b],
                              sss[b]).wait()

    load_idx(0, 0, 0)
    load_idx(1, 1, 0)
    drain_idx(0, 0)
    fire_gathers(0, 0)
    drain_idx(1, 0)
    fire_gathers(1, 0)

    def outer(g, carry):
        for k in range(4):
            b = k % 2
            p = k // 2
            pn = 1 - p
            w = g * 4 + k
            if k < 2:
                load_idx(w + 2, b, pn)
            else:
                @pl.when(g < NGRP // 2 - 1)
                def _():
                    load_idx(w + 2, b, pn)
            drain_gathers(b)
            fire_scatters(b, p)
            drain_scatters(b)
            if k < 2:
                drain_idx(b, pn)
                fire_gathers(b, pn)
            else:
                @pl.when(g < NGRP // 2 - 1)
                def _():
                    drain_idx(b, pn)
                    fire_gathers(b, pn)
        return carry

    lax.fori_loop(0, NGRP // 2, outer, 0)
    plsc.subcore_barrier()

    def wb(dst):
        def go():
            for i in range(4):
                r0 = t * RPT + i * WB
                pltpu.sync_copy(acc_sh.at[pl.ds(r0, WB)], dst.at[pl.ds(r0, WB)])
        return go
    pl.when(c == 0)(wb(o0))
    pl.when(c == 1)(wb(o1))
    if with_deg:
        def wbd(dst):
            def go():
                pltpu.sync_copy(deg_sh.at[pl.ds(t * RPT, RPT)], dv_v)
                pltpu.sync_copy(dv_v, dst.at[pl.ds(t * RPT, RPT)])
            return go
        pl.when(c == 0)(wbd(dj_o))
        pl.when(c == 1)(wbd(du_o))


def _make_segsum(with_deg):
    out_type = [jax.ShapeDtypeStruct((NPAD, HALF), jnp.float32)] * 2
    ring = [pltpu.VMEM((2 * KS2, W), jnp.int32)] * 4 \
        + [pltpu.VMEM((KS2 * W, HALF), jnp.float32)] * 2
    sems = [pltpu.SemaphoreType.DMA] * 6
    if with_deg:
        out_type = out_type + [jax.ShapeDtypeStruct((NPAD,), jnp.float32)] * 2
        scratch = ring + [pltpu.VMEM((KS2, W), jnp.int32)] * 4 \
            + [pltpu.VMEM((W,), jnp.float32),
               pltpu.VMEM((RPT,), jnp.float32),
               pltpu.VMEM_SHARED((NPAD, HALF), jnp.float32),
               pltpu.VMEM_SHARED((NPAD,), jnp.float32)] + sems
    else:
        scratch = ring + [pltpu.VMEM_SHARED((NPAD, HALF), jnp.float32)] + sems
    return pl.kernel(functools.partial(_segsum_body, with_deg),
                     out_type=out_type, mesh=_MESH, scratch_types=scratch,
                     compiler_params=pltpu.CompilerParams(
                         use_tc_tiling_on_sc=False))


_segsum = _make_segsum(False)
_segsum_deg = _make_segsum(True)


def _gathersel_body(*refs):
    (au0h, au1h, nu0h, nu1h, aj0h, aj1h, nj0h, nj1h, du_h, dj_h, ui_h, ji_h,
     oau0, oau1, onu0, onu1, oaj0, oaj1, onj0, onj1, odu, odj,
     iu_v, ij_v, b0, b1, b2, b3, b4, b5, b6, b7, du_v, dj_v, sem) = refs
    c = lax.axis_index("c")
    t = lax.axis_index("s")
    base = (t * NSC + c) * W
    pltpu.sync_copy(ui_h.at[pl.ds(base, W)], iu_v)
    pltpu.sync_copy(ji_h.at[pl.ds(base, W)], ij_v)
    rows = [(au0h, b0, oau0, iu_v), (au1h, b1, oau1, iu_v),
            (nu0h, b2, onu0, iu_v), (nu1h, b3, onu1, iu_v),
            (aj0h, b4, oaj0, ij_v), (aj1h, b5, oaj1, ij_v),
            (nj0h, b6, onj0, ij_v), (nj1h, b7, onj1, ij_v)]
    descs = [pltpu.async_copy(tab.at[iv], buf, sem)
             for tab, buf, _, iv in rows]
    descs.append(pltpu.async_copy(du_h.at[iu_v], du_v, sem))
    descs.append(pltpu.async_copy(dj_h.at[ij_v], dj_v, sem))
    for d in descs:
        d.wait()
    for _, buf, out, _ in rows:
        pltpu.sync_copy(buf, out.at[pl.ds(base, W)])
    pltpu.sync_copy(du_v, odu.at[pl.ds(base, W)])
    pltpu.sync_copy(dj_v, odj.at[pl.ds(base, W)])


_gathersel = pl.kernel(
    _gathersel_body,
    out_type=[jax.ShapeDtypeStruct((BSEL, HALF), jnp.float32)] * 8
             + [jax.ShapeDtypeStruct((BSEL,), jnp.float32)] * 2,
    mesh=_MESH,
    scratch_types=[pltpu.VMEM((W,), jnp.int32)] * 2
                  + [pltpu.VMEM((W, HALF), jnp.float32)] * 8
                  + [pltpu.VMEM((W,), jnp.float32)] * 2
                  + [pltpu.SemaphoreType.DMA],
    compiler_params=pltpu.CompilerParams(use_tc_tiling_on_sc=False))


def kernel(x_user, x_job, edge_index, user_indices, job_indices,
           W_emb_user, b_emb_user, W_emb_job, b_emb_job,
           Wl0_uj, bl0_uj, Wr0_uj, Wl0_ju, bl0_ju, Wr0_ju,
           Wl1_uj, bl1_uj, Wr1_uj, Wl1_ju, bl1_ju, Wr1_ju,
           W_pred, b_pred):
    src = edge_index[0]
    dst = edge_index[1]

    hu0, hu1 = _embed(x_user, W_emb_user, b_emb_user)
    hj0, hj1 = _embed(x_job, W_emb_job, b_emb_job)

    pad_n = EPAD - EDGES
    ar = jnp.arange(pad_n, dtype=jnp.int32)
    pad_g = ar % 128
    pad_s = NJ + ar % (NPAD - NJ)
    src_g = jnp.concatenate([src, pad_g]).reshape(EROWS, W)
    dst_s = jnp.concatenate([dst, pad_s]).reshape(EROWS, W)
    dst_g = jnp.concatenate([dst, pad_g]).reshape(EROWS, W)
    src_s = jnp.concatenate([src, pad_s]).reshape(EROWS, W)
    idxd = jnp.concatenate([dst_s, src_s], axis=0)
    zrow = jnp.zeros((WB, HALF), jnp.float32)
    zdeg = jnp.zeros((RPT,), jnp.float32)

    def pack(g_rows, s_rows):
        nwt = EROWS // KS2
        both = jnp.concatenate([g_rows.reshape(nwt, KS2, W),
                                s_rows.reshape(nwt, KS2, W)], axis=1)
        return both.reshape(nwt * 2 * KS2, W)

    idx_fw = pack(src_g, dst_s)
    idx_bw = pack(dst_g, src_s)

    aj0, aj1, deg_j, deg_u = _segsum_deg(hu0, hu1, idx_fw, idxd,
                                         zrow, zdeg)
    au0, au1 = _segsum(hj0, hj1, idx_bw, zrow)
    nj0, nj1 = _combine(aj0, aj1, deg_j, hj0, hj1, Wl0_uj, bl0_uj, Wr0_uj)
    nu0, nu1 = _combine(au0, au1, deg_u, hu0, hu1, Wl0_ju, bl0_ju, Wr0_ju)

    aj0, aj1 = _segsum(nu0, nu1, idx_fw, zrow)
    au0, au1 = _segsum(nj0, nj1, idx_bw, zrow)

    wp_u = W_pred[:DH, 0]
    wp_j = W_pred[DH:, 0]
    v1 = Wl1_ju @ wp_u
    v2 = Wr1_ju @ wp_u
    v3 = Wl1_uj @ wp_j
    v4 = Wr1_uj @ wp_j
    c = bl1_ju @ wp_u + bl1_uj @ wp_j + b_pred[0]
    vpack = jnp.stack([v1[:HALF], v1[HALF:], v2[:HALF], v2[HALF:],
                       v3[:HALF], v3[HALF:], v4[:HALF], v4[HALF:]], axis=0)

    (au0s, au1s, ue0, ue1, aj0s, aj1s, je0, je1, dus, djs) = _gathersel(
        au0, au1, nu0, nu1, aj0, aj1, nj0, nj1, deg_u, deg_j,
        user_indices, job_indices)

    preds = _pred(au0s, au1s, dus, ue0, ue1, aj0s, aj1s, djs, je0, je1,
                  vpack, c)
    return preds[:, 0]


# speedup vs baseline: 2.3864x; 1.0002x over previous
import functools

import jax
import jax.numpy as jnp
from jax import lax
from jax.experimental import pallas as pl
from jax.experimental.pallas import tpu as pltpu
from jax.experimental.pallas import tpu_sc as plsc

NU = 50000
NJ = 50000
EDGES = 800000
DF = 128
DE = 64
DH = 64
BSEL = 4096
HALF = 32

NSC = 2
NTILE = 16
W = 128
KSUB = 4
SUPS = 98
KS2 = 2
NWIN = SUPS * KSUB // KS2
NGRP = NWIN // 2
TILE_E = SUPS * KSUB * W
EPAD = NTILE * TILE_E
EROWS = EPAD // W
NPAD = 50176
RPT = NPAD // NTILE
WB = RPT // 4

_MESH = plsc.VectorSubcoreMesh(core_axis_name="c", subcore_axis_name="s",
                               num_cores=NSC, num_subcores=NTILE)


def _embed_block(x_ref, w_ref, b_ref, h0_ref, h1_ref):
    h = jnp.dot(x_ref[...], w_ref[...], preferred_element_type=jnp.float32)
    h = h + b_ref[...]
    h0_ref[...] = h[:, :HALF]
    h1_ref[...] = h[:, HALF:]


def _embed(x, w, b, rows=1000):
    n = x.shape[0]
    return pl.pallas_call(
        _embed_block,
        grid=(n // rows,),
        in_specs=[pl.BlockSpec((rows, DF), lambda i: (i, 0)),
                  pl.BlockSpec((DF, DE), lambda i: (0, 0)),
                  pl.BlockSpec((1, DE), lambda i: (0, 0))],
        out_specs=[pl.BlockSpec((rows, HALF), lambda i: (i, 0)),
                   pl.BlockSpec((rows, HALF), lambda i: (i, 0))],
        out_shape=[jax.ShapeDtypeStruct((n, HALF), jnp.float32)] * 2,
    )(x, w, b.reshape(1, DE))


def _combine_block(a0_ref, a1_ref, d_ref, h0_ref, h1_ref,
                   wl_ref, bl_ref, wr_ref, o0_ref, o1_ref):
    r = 1.0 / jnp.maximum(d_ref[...], 1.0)
    a = jnp.concatenate([a0_ref[...] * r, a1_ref[...] * r], axis=1)
    h = jnp.concatenate([h0_ref[...], h1_ref[...]], axis=1)
    o = jnp.dot(a, wl_ref[...], preferred_element_type=jnp.float32)
    o = o + bl_ref[...]
    o = o + jnp.dot(h, wr_ref[...], preferred_element_type=jnp.float32)
    o = jnp.maximum(o, 0.0)
    o0_ref[...] = o[:, :HALF]
    o1_ref[...] = o[:, HALF:]


def _combine(a0, a1, deg, h0, h1, wl, bl, wr, rows=1000):
    n = h0.shape[0]
    return pl.pallas_call(
        _combine_block,
        grid=(n // rows,),
        in_specs=[pl.BlockSpec((rows, HALF), lambda i: (i, 0)),
                  pl.BlockSpec((rows, HALF), lambda i: (i, 0)),
                  pl.BlockSpec((rows, 1), lambda i: (i, 0)),
                  pl.BlockSpec((rows, HALF), lambda i: (i, 0)),
                  pl.BlockSpec((rows, HALF), lambda i: (i, 0)),
                  pl.BlockSpec((DE, DH), lambda i: (0, 0)),
                  pl.BlockSpec((1, DH), lambda i: (0, 0)),
                  pl.BlockSpec((DE, DH), lambda i: (0, 0))],
        out_specs=[pl.BlockSpec((rows, HALF), lambda i: (i, 0)),
                   pl.BlockSpec((rows, HALF), lambda i: (i, 0))],
        out_shape=[jax.ShapeDtypeStruct((n, HALF), jnp.float32)] * 2,
    )(a0, a1, deg.reshape(n, 1) if deg.shape[0] == n else deg[:n].reshape(n, 1),
      h0, h1, wl, bl.reshape(1, DH), wr)


def _pred_block(au0, au1, du, ue0, ue1, aj0, aj1, dj, je0, je1,
                vp_ref, c_ref, o_ref):
    vp = vp_ref[...]
    ru = 1.0 / jnp.maximum(du[...], 1.0)
    rj = 1.0 / jnp.maximum(dj[...], 1.0)
    s = jnp.sum(au0[...] * vp[0:1] + au1[...] * vp[1:2], axis=1, keepdims=True) * ru
    s = s + jnp.sum(ue0[...] * vp[2:3] + ue1[...] * vp[3:4], axis=1, keepdims=True)
    s = s + jnp.sum(aj0[...] * vp[4:5] + aj1[...] * vp[5:6], axis=1, keepdims=True) * rj
    s = s + jnp.sum(je0[...] * vp[6:7] + je1[...] * vp[7:8], axis=1, keepdims=True)
    o_ref[...] = s + c_ref[...]


def _pred(au0, au1, du, ue0, ue1, aj0, aj1, dj, je0, je1, vpack, c):
    n = au0.shape[0]
    half_spec = pl.BlockSpec((n, HALF), lambda: (0, 0))
    one_spec = pl.BlockSpec((n, 1), lambda: (0, 0))
    return pl.pallas_call(
        _pred_block,
        in_specs=[half_spec, half_spec, one_spec, half_spec, half_spec,
                  half_spec, half_spec, one_spec, half_spec, half_spec,
                  pl.BlockSpec((8, HALF), lambda: (0, 0)),
                  pl.BlockSpec((1, 1), lambda: (0, 0))],
        out_specs=pl.BlockSpec((n, 1), lambda: (0, 0)),
        out_shape=jax.ShapeDtypeStruct((n, 1), jnp.float32),
    )(au0, au1, du.reshape(n, 1), ue0, ue1, aj0, aj1, dj.reshape(n, 1),
      je0, je1, vpack, c.reshape(1, 1))


def _segsum_body(with_deg, *refs):
    if with_deg:
        (t0h, t1h, idxgs_h, idxd_h, zrow_h, zdeg_h,
         o0, o1, dj_o, du_o,
         ix00, ix01, ix10, ix11, r0, r1,
         id00, id01, id10, id11, ones_v, dv_v,
         acc_sh, deg_sh, sg0, sg1, ss0, ss1, si0, si1) = refs
        ids = ((id00, id01), (id10, id11))
    else:
        (t0h, t1h, idxgs_h, zrow_h,
         o0, o1,
         ix00, ix01, ix10, ix11, r0, r1,
         acc_sh, sg0, sg1, ss0, ss1, si0, si1) = refs
    ixs = ((ix00, ix01), (ix10, ix11))
    rows = (r0, r1)
    sgs = (sg0, sg1)
    sss = (ss0, ss1)
    sis = (si0, si1)
    c = lax.axis_index("c")
    t = lax.axis_index("s")

    for i in range(4):
        pltpu.sync_copy(zrow_h, acc_sh.at[pl.ds(t * RPT + i * WB, WB)])
    if with_deg:
        pltpu.sync_copy(zdeg_h, dv_v)
        pltpu.sync_copy(dv_v, deg_sh.at[pl.ds(t * RPT, RPT)])
        for i in range(W // 16):
            ones_v[pl.ds(i * 16, 16)] = jnp.full((16,), 1.0, jnp.float32)
    plsc.subcore_barrier()

    tbase = t * NWIN

    def load_idx(w, b, p):
        gw = tbase + w
        pltpu.async_copy(idxgs_h.at[pl.ds(gw * 2 * KS2, 2 * KS2)],
                         ixs[b][p], sis[b])
        if with_deg:
            pltpu.async_copy(idxd_h.at[pl.ds(c * EROWS + gw * KS2, KS2)],
                             ids[b][p], sis[b])

    def drain_idx(b, p):
        pltpu.make_async_copy(idxgs_h.at[pl.ds(0, 2 * KS2)], ixs[b][p],
                              sis[b]).wait()
        if with_deg:
            pltpu.make_async_copy(idxd_h.at[pl.ds(0, KS2)], ids[b][p],
                                  sis[b]).wait()

    def fire_gathers(b, p):
        def fire(tab):
            def go():
                for j in range(KS2):
                    pltpu.async_copy(tab.at[ixs[b][p].at[j]],
                                     rows[b].at[pl.ds(j * W, W)], sgs[b])
            return go
        pl.when(c == 0)(fire(t0h))
        pl.when(c == 1)(fire(t1h))

    def drain_gathers(b):
        pltpu.make_async_copy(t0h.at[pl.ds(0, KS2 * W)], rows[b],
                              sgs[b]).wait()

    def fire_scatters(b, p):
        for j in range(KS2):
            pltpu.async_copy(rows[b].at[pl.ds(j * W, W)],
                             acc_sh.at[ixs[b][p].at[KS2 + j]], sss[b],
                             add=True)
        if with_deg:
            for j in range(KS2):
                pltpu.async_copy(ones_v, deg_sh.at[ids[b][p].at[j]],
                                 sss[b], add=True)

    def drain_scatters(b):
        pltpu.make_async_copy(t0h.at[pl.ds(0, KS2 * W)], rows[b],
                              sss[b]).wait()
        if with_deg:
            for j in range(KS2):
                pltpu.make_async_copy(dj_o.at[pl.ds(0, W)], ones_v,
                                      sss[b]).wait()

    load_idx(0, 0, 0)
    load_idx(1, 1, 0)
    drain_idx(0, 0)
    fire_gathers(0, 0)
    drain_idx(1, 0)
    fire_gathers(1, 0)

    def outer(g, carry):
        for k in range(4):
            b = k % 2
            p = k // 2
            pn = 1 - p
            w = g * 4 + k
            if k < 2:
                load_idx(w + 2, b, pn)
            else:
                @pl.when(g < NGRP // 2 - 1)
                def _():
                    load_idx(w + 2, b, pn)
            drain_gathers(b)
            fire_scatters(b, p)
            drain_scatters(b)
            if k < 2:
                drain_idx(b, pn)
                fire_gathers(b, pn)
            else:
                @pl.when(g < NGRP // 2 - 1)
                def _():
                    drain_idx(b, pn)
                    fire_gathers(b, pn)
        return carry

    lax.fori_loop(0, NGRP // 2, outer, 0)
    plsc.subcore_barrier()

    def wb(dst):
        def go():
            for i in range(4):
                r0 = t * RPT + i * WB
                pltpu.sync_copy(acc_sh.at[pl.ds(r0, WB)], dst.at[pl.ds(r0, WB)])
        return go
    pl.when(c == 0)(wb(o0))
    pl.when(c == 1)(wb(o1))
    if with_deg:
        def wbd(dst):
            def go():
                pltpu.sync_copy(deg_sh.at[pl.ds(t * RPT, RPT)], dv_v)
                pltpu.sync_copy(dv_v, dst.at[pl.ds(t * RPT, RPT)])
            return go
        pl.when(c == 0)(wbd(dj_o))
        pl.when(c == 1)(wbd(du_o))


def _make_segsum(with_deg):
    out_type = [jax.ShapeDtypeStruct((NPAD, HALF), jnp.float32)] * 2
    ring = [pltpu.VMEM((2 * KS2, W), jnp.int32)] * 4 \
        + [pltpu.VMEM((KS2 * W, HALF), jnp.float32)] * 2
    sems = [pltpu.SemaphoreType.DMA] * 6
    if with_deg:
        out_type = out_type + [jax.ShapeDtypeStruct((NPAD,), jnp.float32)] * 2
        scratch = ring + [pltpu.VMEM((KS2, W), jnp.int32)] * 4 \
            + [pltpu.VMEM((W,), jnp.float32),
               pltpu.VMEM((RPT,), jnp.float32),
               pltpu.VMEM_SHARED((NPAD, HALF), jnp.float32),
               pltpu.VMEM_SHARED((NPAD,), jnp.float32)] + sems
    else:
        scratch = ring + [pltpu.VMEM_SHARED((NPAD, HALF), jnp.float32)] + sems
    return pl.kernel(functools.partial(_segsum_body, with_deg),
                     out_type=out_type, mesh=_MESH, scratch_types=scratch,
                     compiler_params=pltpu.CompilerParams(
                         use_tc_tiling_on_sc=False))


_segsum = _make_segsum(False)
_segsum_deg = _make_segsum(True)


def _gathersel_body(*refs):
    (au0h, au1h, nu0h, nu1h, aj0h, aj1h, nj0h, nj1h, du_h, dj_h, ui_h, ji_h,
     oau0, oau1, onu0, onu1, oaj0, oaj1, onj0, onj1, odu, odj,
     iu_v, ij_v, b0, b1, b2, b3, b4, b5, b6, b7, du_v, dj_v, sem) = refs
    c = lax.axis_index("c")
    t = lax.axis_index("s")
    base = (t * NSC + c) * W
    pltpu.sync_copy(ui_h.at[pl.ds(base, W)], iu_v)
    pltpu.sync_copy(ji_h.at[pl.ds(base, W)], ij_v)
    rows = [(au0h, b0, oau0, iu_v), (au1h, b1, oau1, iu_v),
            (nu0h, b2, onu0, iu_v), (nu1h, b3, onu1, iu_v),
            (aj0h, b4, oaj0, ij_v), (aj1h, b5, oaj1, ij_v),
            (nj0h, b6, onj0, ij_v), (nj1h, b7, onj1, ij_v)]
    descs = [pltpu.async_copy(tab.at[iv], buf, sem)
             for tab, buf, _, iv in rows]
    descs.append(pltpu.async_copy(du_h.at[iu_v], du_v, sem))
    descs.append(pltpu.async_copy(dj_h.at[ij_v], dj_v, sem))
    for d in descs:
        d.wait()
    for _, buf, out, _ in rows:
        pltpu.sync_copy(buf, out.at[pl.ds(base, W)])
    pltpu.sync_copy(du_v, odu.at[pl.ds(base, W)])
    pltpu.sync_copy(dj_v, odj.at[pl.ds(base, W)])


_gathersel = pl.kernel(
    _gathersel_body,
    out_type=[jax.ShapeDtypeStruct((BSEL, HALF), jnp.float32)] * 8
             + [jax.ShapeDtypeStruct((BSEL,), jnp.float32)] * 2,
    mesh=_MESH,
    scratch_types=[pltpu.VMEM((W,), jnp.int32)] * 2
                  + [pltpu.VMEM((W, HALF), jnp.float32)] * 8
                  + [pltpu.VMEM((W,), jnp.float32)] * 2
                  + [pltpu.SemaphoreType.DMA],
    compiler_params=pltpu.CompilerParams(use_tc_tiling_on_sc=False))


def kernel(x_user, x_job, edge_index, user_indices, job_indices,
           W_emb_user, b_emb_user, W_emb_job, b_emb_job,
           Wl0_uj, bl0_uj, Wr0_uj, Wl0_ju, bl0_ju, Wr0_ju,
           Wl1_uj, bl1_uj, Wr1_uj, Wl1_ju, bl1_ju, Wr1_ju,
           W_pred, b_pred):
    src = edge_index[0]
    dst = edge_index[1]

    hu0, hu1 = _embed(x_user, W_emb_user, b_emb_user)
    hj0, hj1 = _embed(x_job, W_emb_job, b_emb_job)

    pad_n = EPAD - EDGES
    ar = jnp.arange(pad_n, dtype=jnp.int32)
    pad_g = ar % 128
    pad_s = NJ + ar % (NPAD - NJ)
    src_g = jnp.concatenate([src, pad_g]).reshape(EROWS, W)
    dst_s = jnp.concatenate([dst, pad_s]).reshape(EROWS, W)
    dst_g = jnp.concatenate([dst, pad_g]).reshape(EROWS, W)
    src_s = jnp.concatenate([src, pad_s]).reshape(EROWS, W)
    idxd = jnp.concatenate([dst_s, src_s], axis=0)
    zrow = jnp.zeros((WB, HALF), jnp.float32)
    zdeg = jnp.zeros((RPT,), jnp.float32)

    def pack(g_rows, s_rows):
        nwt = EROWS // KS2
        both = jnp.concatenate([g_rows.reshape(nwt, KS2, W),
                                s_rows.reshape(nwt, KS2, W)], axis=1)
        return both.reshape(nwt * 2 * KS2, W)

    idx_fw = pack(src_g, dst_s)
    idx_bw = pack(dst_g, src_s)

    aj0, aj1, deg_j, deg_u = _segsum_deg(hu0, hu1, idx_fw, idxd,
                                         zrow, zdeg)
    au0, au1 = _segsum(hj0, hj1, idx_bw, zrow)
    nj0, nj1 = _combine(aj0, aj1, deg_j, hj0, hj1, Wl0_uj, bl0_uj, Wr0_uj)
    nu0, nu1 = _combine(au0, au1, deg_u, hu0, hu1, Wl0_ju, bl0_ju, Wr0_ju)

    aj0, aj1 = _segsum(nu0, nu1, idx_fw, zrow)
    au0, au1 = _segsum(nj0, nj1, idx_bw, zrow)

    wp_u = W_pred[:DH, 0]
    wp_j = W_pred[DH:, 0]
    v1 = Wl1_ju @ wp_u
    v2 = Wr1_ju @ wp_u
    v3 = Wl1_uj @ wp_j
    v4 = Wr1_uj @ wp_j
    c = bl1_ju @ wp_u + bl1_uj @ wp_j + b_pred[0]
    vpack = jnp.stack([v1[:HALF], v1[HALF:], v2[:HALF], v2[HALF:],
                       v3[:HALF], v3[HALF:], v4[:HALF], v4[HALF:]], axis=0)

    (au0s, au1s, ue0, ue1, aj0s, aj1s, je0, je1, dus, djs) = _gathersel(
        au0, au1, nu0, nu1, aj0, aj1, nj0, nj1, deg_u, deg_j,
        user_indices, job_indices)

    preds = _pred(au0s, au1s, dus, ue0, ue1, aj0s, aj1s, djs, je0, je1,
                  vpack, c)
    return preds[:, 0]
